# baseline (speedup 1.0000x reference)
import functools

import jax
import jax.numpy as jnp
from jax import lax
from jax.experimental import pallas as pl
from jax.experimental.pallas import tpu as pltpu
from jax.experimental.pallas import tpu_sc as plsc

_B, _C, _N, _M = 2, 256, 1024, 1024
_P = _N + _M
_K = 16
_EPS = 1e-5
_R = 256
_TPB = _P // 8 * (_P // 128)


def _dist_body(f_blk_ref, f_all_ref, d_ref):
    fb = f_blk_ref[0]
    fa = f_all_ref[0]
    g = lax.dot_general(fb, fa, (((0,), (0,)), ((), ())),
                        preferred_element_type=jnp.float32)
    c2 = jnp.sum(fa * fa, axis=0)
    d = c2[None, :] - 2.0 * g
    dt = d.reshape(_R // 8, 8, _P // 128, 128).transpose(0, 2, 1, 3)
    d_ref[0] = dt.reshape(_R // 8 * (_P // 128), 8, 128)


def _dist(ft):
    return pl.pallas_call(
        _dist_body,
        grid=(_B, _P // _R),
        in_specs=[
            pl.BlockSpec((1, _C, _R), lambda b, i: (b, 0, i)),
            pl.BlockSpec((1, _C, _P), lambda b, i: (b, 0, 0)),
        ],
        out_specs=pl.BlockSpec((1, _R // 8 * (_P // 128), 8, 128),
                               lambda b, i: (b, i, 0, 0)),
        out_shape=jax.ShapeDtypeStruct((_B, _TPB, 8, 128), jnp.float32),
    )(ft, ft)


def _sc_topk(dt):
    mesh = plsc.VectorSubcoreMesh(core_axis_name="c", subcore_axis_name="s")

    @functools.partial(
        pl.kernel,
        out_type=jax.ShapeDtypeStruct((_B, _P, 2 * _K), jnp.int32),
        mesh=mesh,
        scratch_types=[
            pltpu.VMEM((16, 8, 128), jnp.float32),
            pltpu.VMEM((16, 8, 128), jnp.float32),
            pltpu.VMEM((128, 2 * _K), jnp.int32),
            pltpu.SemaphoreType.DMA,
            pltpu.SemaphoreType.DMA,
        ],
        compiler_params=pltpu.CompilerParams(
            use_tc_tiling_on_sc=False, needs_layout_passes=False),
    )
    def tk(d_hbm, out_hbm, d_v0, d_v1, out_v, sem0, sem1):
        cid = lax.axis_index("c")
        sid = lax.axis_index("s")
        wid = sid * 2 + cid
        b = wid // 16
        rb0 = (wid % 16) * 16
        iota = lax.broadcasted_iota(jnp.int32, (16,), 0)

        def dcopy(blk, buf, sem):
            return pltpu.make_async_copy(
                d_hbm.at[b, pl.ds((rb0 + blk) * 16, 16)], buf, sem)

        def process(buf, blk):
            for grp in range(2):
                ks, vs = [], []
                for r in range(4 * grp, 4 * grp + 4):
                    for h in range(2):
                        k0 = buf[h * 8, r, pl.ds(0, 16)]
                        sk, sv = plsc.sort_key_val(k0, iota + h * 1024)
                        ks.append(sk)
                        vs.append(sv)

                def chunk(c, kv):
                    ks, vs = kv
                    nks, nvs = [], []
                    for hh in range(2):
                        c16 = hh * 64 + c
                        ci = c16 // 8
                        co = (c16 % 8) * 16
                        base = hh * 1024 + c * 16
                        for rr in range(4):
                            r = 4 * grp + rr
                            j = rr * 2 + hh
                            kc = buf[ci, r, pl.ds(co, 16)]
                            kcd, icd = plsc.sort_key_val(
                                kc, iota + base, descending=True)
                            sel = kcd < ks[j]
                            km = jnp.where(sel, kcd, ks[j])
                            vm = jnp.where(sel, icd, vs[j])
                            km, vm = plsc.sort_key_val(km, vm)
                            nks.append(km)
                            nvs.append(vm)
                    nks = [nks[hh * 4 + rr] for rr in range(4) for hh in range(2)]
                    nvs = [nvs[hh * 4 + rr] for rr in range(4) for hh in range(2)]
                    return tuple(nks), tuple(nvs)

                ks, vs = lax.fori_loop(1, 64, chunk, (tuple(ks), tuple(vs)))
                for rr in range(4):
                    r = 4 * grp + rr
                    for h in range(2):
                        out_v[blk * 8 + r, pl.ds(h * 16, 16)] = vs[rr * 2 + h]

        dcopy(0, d_v0, sem0).start()
        dcopy(1, d_v1, sem1).start()

        def pair(i, carry):
            blk0 = 2 * i
            dcopy(blk0, d_v0, sem0).wait()
            process(d_v0, blk0)

            @pl.when(i < 7)
            def _():
                dcopy(blk0 + 2, d_v0, sem0).start()

            dcopy(blk0 + 1, d_v1, sem1).wait()
            process(d_v1, blk0 + 1)

            @pl.when(i < 7)
            def _():
                dcopy(blk0 + 3, d_v1, sem1).start()

            return carry

        lax.fori_loop(0, 8, pair, 0)
        pltpu.sync_copy(out_v, out_hbm.at[b, pl.ds((wid % 16) * 128, 128)])

    return tk(dt)


_TILES_PER_B = 16
_CH = _C // _TILES_PER_B
_PH = _P // 2
_PC = 16


def _gather_max(ftp, idxt):
    mesh = plsc.VectorSubcoreMesh(core_axis_name="c", subcore_axis_name="s")

    @functools.partial(
        pl.kernel,
        out_type=jax.ShapeDtypeStruct((_B, _P, _C), jnp.float32),
        mesh=mesh,
        scratch_types=[
            pltpu.VMEM((_P, _CH), jnp.float32),
            pltpu.VMEM((_P, 2 * _K), jnp.int32),
            pltpu.VMEM((_PH, _CH), jnp.float32),
        ],
        compiler_params=pltpu.CompilerParams(
            use_tc_tiling_on_sc=False, needs_layout_passes=False),
    )
    def gm(ftp_hbm, idx_hbm, out_hbm, f_v, idx_v, out_v):
        cid = lax.axis_index("c")
        sid = lax.axis_index("s")
        wid = sid * 2 + cid
        b = wid // _TILES_PER_B
        c0 = (wid % _TILES_PER_B) * _CH
        pltpu.sync_copy(ftp_hbm.at[b, :, pl.ds(c0, _CH)], f_v)
        pltpu.sync_copy(idx_hbm.at[b], idx_v)
        for h in range(2):
            def chunk(i, carry):
                p0 = h * _PH + i * _PC
                ivs = [idx_v[p0 + k, pl.ds(j * 16, 16)]
                       for k in range(_PC) for j in range(2)]
                for k in range(_PC):
                    iv0 = ivs[2 * k]
                    iv1 = ivs[2 * k + 1]
                    rows = [f_v[iv0[l], :] for l in range(16)]
                    rows += [f_v[iv1[l], :] for l in range(16)]
                    while len(rows) > 1:
                        rows = [jnp.maximum(rows[2 * t], rows[2 * t + 1])
                                for t in range(len(rows) // 2)]
                    out_v[i * _PC + k, :] = rows[0]
                return carry

            lax.fori_loop(0, _PH // _PC, chunk, 0)
            pltpu.sync_copy(out_v, out_hbm.at[b, pl.ds(h * _PH, _PH),
                                              pl.ds(c0, _CH)])

    return gm(ftp, idxt)


def _conv_body(ft_ref, mx_ref, wa_ref, wb_ref, b_ref, g_ref, be_ref,
               out_ref, outp_ref, split=False):
    wa = wa_ref[...]
    wb = wb_ref[...]
    wd = wa - wb
    bias = b_ref[:]
    outs = []
    s1 = jnp.zeros((_C, 1), jnp.float32)
    for b in range(_B):
        o = (jnp.dot(wd, ft_ref[b], preferred_element_type=jnp.float32)
             + lax.dot_general(wb, mx_ref[b], (((1,), (1,)), ((), ())),
                               preferred_element_type=jnp.float32))
        o = o + bias[:, None]
        outs.append(o)
        s1 = s1 + jnp.sum(o, axis=1, keepdims=True)
    mean = s1 * (1.0 / (_B * _P))
    s2 = jnp.zeros((_C, 1), jnp.float32)
    for b in range(_B):
        ctr = outs[b] - mean
        outs[b] = ctr
        s2 = s2 + jnp.sum(ctr * ctr, axis=1, keepdims=True)
    var = s2 * (1.0 / (_B * _P))
    scale = lax.rsqrt(var + _EPS) * g_ref[:][:, None]
    for b in range(_B):
        o = outs[b] * scale + be_ref[:][:, None]
        o = jax.nn.gelu(o) + ft_ref[b]
        if split:
            out_ref[b] = o[:, :_N]
            outp_ref[b] = o[:, _N:]
        else:
            out_ref[b] = o
            if outp_ref is not None:
                outp_ref[b] = o.T


def _conv(ft, mx, w, b, g, be, want_pc):
    wint = w.reshape(_C, _C, 2)
    if want_pc:
        body = _conv_body
        out_shape = (jax.ShapeDtypeStruct((_B, _C, _P), jnp.float32),
                     jax.ShapeDtypeStruct((_B, _P, _C), jnp.float32))
    else:
        body = functools.partial(_conv_body, split=True)
        out_shape = (jax.ShapeDtypeStruct((_B, _C, _N), jnp.float32),
                     jax.ShapeDtypeStruct((_B, _C, _M), jnp.float32))
    return pl.pallas_call(
        body,
        out_shape=out_shape,
    )(ft, mx, wint[:, :, 0], wint[:, :, 1], b, g, be)


def kernel(x, y, W1, b1, g1, be1, W2, b2, g2, be2):
    ft0 = jnp.concatenate([x[..., 0], y[..., 0]], axis=2)
    ftp0 = jnp.swapaxes(ft0, 1, 2)
    dt = _dist(ft0)
    idxt = _sc_topk(dt)
    mx0 = _gather_max(ftp0, idxt)
    ft1, ftp1 = _conv(ft0, mx0, W1, b1, g1, be1, True)
    mx1 = _gather_max(ftp1, idxt)
    ox, oy = _conv(ft1, mx1, W2, b2, g2, be2, False)
    return ox[..., None], oy[..., None]


# speedup vs baseline: 1.0699x; 1.0699x over previous
import functools

import jax
import jax.numpy as jnp
from jax import lax
from jax.experimental import pallas as pl
from jax.experimental.pallas import tpu as pltpu
from jax.experimental.pallas import tpu_sc as plsc

_B, _C, _N, _M = 2, 256, 1024, 1024
_P = _N + _M
_K = 16
_EPS = 1e-5
_R = 256
_TPB = _P // 8 * (_P // 128)


def _dist_body(f_blk_ref, f_all_ref, d_ref):
    fb = f_blk_ref[0]
    fa = f_all_ref[0]
    g = lax.dot_general(fb, fa, (((0,), (0,)), ((), ())),
                        preferred_element_type=jnp.float32)
    c2 = jnp.sum(fa * fa, axis=0)
    d = c2[None, :] - 2.0 * g
    dt = d.reshape(_R // 8, 8, _P // 128, 128).transpose(0, 2, 1, 3)
    d_ref[0] = dt.reshape(_R // 8 * (_P // 128), 8, 128)


def _dist(ft):
    return pl.pallas_call(
        _dist_body,
        grid=(_B, _P // _R),
        in_specs=[
            pl.BlockSpec((1, _C, _R), lambda b, i: (b, 0, i)),
            pl.BlockSpec((1, _C, _P), lambda b, i: (b, 0, 0)),
        ],
        out_specs=pl.BlockSpec((1, _R // 8 * (_P // 128), 8, 128),
                               lambda b, i: (b, i, 0, 0)),
        out_shape=jax.ShapeDtypeStruct((_B, _TPB, 8, 128), jnp.float32),
    )(ft, ft)


def _sc_topk(dt):
    mesh = plsc.VectorSubcoreMesh(core_axis_name="c", subcore_axis_name="s")

    @functools.partial(
        pl.kernel,
        out_type=jax.ShapeDtypeStruct((_B, _P, 2 * _K), jnp.int32),
        mesh=mesh,
        scratch_types=[
            pltpu.VMEM((16, 8, 128), jnp.float32),
            pltpu.VMEM((16, 8, 128), jnp.float32),
            pltpu.VMEM((128, 2 * _K), jnp.int32),
            pltpu.SemaphoreType.DMA,
            pltpu.SemaphoreType.DMA,
        ],
        compiler_params=pltpu.CompilerParams(
            use_tc_tiling_on_sc=False, needs_layout_passes=False),
    )
    def tk(d_hbm, out_hbm, d_v0, d_v1, out_v, sem0, sem1):
        cid = lax.axis_index("c")
        sid = lax.axis_index("s")
        wid = sid * 2 + cid
        b = wid // 16
        rb0 = (wid % 16) * 16
        iota = lax.broadcasted_iota(jnp.int32, (16,), 0)

        def dcopy(blk, buf, sem):
            return pltpu.make_async_copy(
                d_hbm.at[b, pl.ds((rb0 + blk) * 16, 16)], buf, sem)

        def process(buf, blk):
            for grp in range(2):
                ks, vs = [], []
                for r in range(4 * grp, 4 * grp + 4):
                    for h in range(2):
                        k0 = buf[h * 8, r, pl.ds(0, 16)]
                        sk, sv = plsc.sort_key_val(k0, iota + h * 1024)
                        ks.append(sk)
                        vs.append(sv)

                def chunk(c, kv):
                    ks, vs = kv
                    nks, nvs = [], []
                    for hh in range(2):
                        c16 = hh * 64 + c
                        ci = c16 // 8
                        co = (c16 % 8) * 16
                        base = hh * 1024 + c * 16
                        for rr in range(4):
                            r = 4 * grp + rr
                            j = rr * 2 + hh
                            kc = buf[ci, r, pl.ds(co, 16)]
                            kcd, icd = plsc.sort_key_val(
                                kc, iota + base, descending=True)
                            sel = kcd < ks[j]
                            km = jnp.where(sel, kcd, ks[j])
                            vm = jnp.where(sel, icd, vs[j])
                            km, vm = plsc.sort_key_val(km, vm)
                            nks.append(km)
                            nvs.append(vm)
                    nks = [nks[hh * 4 + rr] for rr in range(4) for hh in range(2)]
                    nvs = [nvs[hh * 4 + rr] for rr in range(4) for hh in range(2)]
                    return tuple(nks), tuple(nvs)

                ks, vs = lax.fori_loop(1, 64, chunk, (tuple(ks), tuple(vs)))
                for rr in range(4):
                    r = 4 * grp + rr
                    for h in range(2):
                        out_v[blk * 8 + r, pl.ds(h * 16, 16)] = vs[rr * 2 + h]

        dcopy(0, d_v0, sem0).start()
        dcopy(1, d_v1, sem1).start()

        def pair(i, carry):
            blk0 = 2 * i
            dcopy(blk0, d_v0, sem0).wait()
            process(d_v0, blk0)

            @pl.when(i < 7)
            def _():
                dcopy(blk0 + 2, d_v0, sem0).start()

            dcopy(blk0 + 1, d_v1, sem1).wait()
            process(d_v1, blk0 + 1)

            @pl.when(i < 7)
            def _():
                dcopy(blk0 + 3, d_v1, sem1).start()

            return carry

        lax.fori_loop(0, 8, pair, 0)
        pltpu.sync_copy(out_v, out_hbm.at[b, pl.ds((wid % 16) * 128, 128)])

    return tk(dt)


_TILES_PER_B = 16
_CH = _C // _TILES_PER_B
_PH = _P // 2
_PC = 16


def _gather_max(ftp, idxt):
    mesh = plsc.VectorSubcoreMesh(core_axis_name="c", subcore_axis_name="s")

    @functools.partial(
        pl.kernel,
        out_type=jax.ShapeDtypeStruct((_B, _P // 8, _C // 128, 8, 128),
                                      jnp.float32),
        mesh=mesh,
        scratch_types=[
            pltpu.VMEM((_P, _CH), jnp.float32),
            pltpu.VMEM((_P, 2 * _K), jnp.int32),
            pltpu.VMEM((_PH // 8, 8, _CH), jnp.float32),
        ],
        compiler_params=pltpu.CompilerParams(
            use_tc_tiling_on_sc=False, needs_layout_passes=False),
    )
    def gm(ftp_hbm, idx_hbm, out_hbm, f_v, idx_v, out_v):
        cid = lax.axis_index("c")
        sid = lax.axis_index("s")
        wid = sid * 2 + cid
        b = wid // _TILES_PER_B
        c0 = (wid % _TILES_PER_B) * _CH
        pltpu.sync_copy(ftp_hbm.at[b, :, pl.ds(c0, _CH)], f_v)
        pltpu.sync_copy(idx_hbm.at[b], idx_v)
        for h in range(2):
            def chunk(i, carry):
                p0 = h * _PH + i * _PC
                ivs = [idx_v[p0 + k, pl.ds(j * 16, 16)]
                       for k in range(_PC) for j in range(2)]
                for k in range(_PC):
                    iv0 = ivs[2 * k]
                    iv1 = ivs[2 * k + 1]
                    m = f_v[iv0[0], :]
                    for l in range(1, 16):
                        m = jnp.maximum(m, f_v[iv0[l], :])
                    for l in range(16):
                        m = jnp.maximum(m, f_v[iv1[l], :])
                    q = i * _PC + k
                    out_v[q // 8, q % 8, :] = m
                return carry

            lax.fori_loop(0, _PH // _PC, chunk, 0)
            pltpu.sync_copy(
                out_v,
                out_hbm.at[b, pl.ds(h * (_PH // 8), _PH // 8), c0 // 128,
                           :, pl.ds(c0 % 128, _CH)])

    return gm(ftp, idxt)


def _conv_body(ft_ref, mx_ref, wa_ref, wb_ref, b_ref, g_ref, be_ref,
               out_ref, outp_ref, split=False):
    wa = wa_ref[...]
    wb = wb_ref[...]
    wd = wa - wb
    bias = b_ref[:]
    outs = []
    s1 = jnp.zeros((_C, 1), jnp.float32)
    for b in range(_B):
        mxp = mx_ref[b].transpose(0, 2, 1, 3).reshape(_P, _C)
        o = (jnp.dot(wd, ft_ref[b], preferred_element_type=jnp.float32)
             + lax.dot_general(wb, mxp, (((1,), (1,)), ((), ())),
                               preferred_element_type=jnp.float32))
        o = o + bias[:, None]
        outs.append(o)
        s1 = s1 + jnp.sum(o, axis=1, keepdims=True)
    mean = s1 * (1.0 / (_B * _P))
    s2 = jnp.zeros((_C, 1), jnp.float32)
    for b in range(_B):
        ctr = outs[b] - mean
        outs[b] = ctr
        s2 = s2 + jnp.sum(ctr * ctr, axis=1, keepdims=True)
    var = s2 * (1.0 / (_B * _P))
    scale = lax.rsqrt(var + _EPS) * g_ref[:][:, None]
    for b in range(_B):
        o = outs[b] * scale + be_ref[:][:, None]
        o = jax.nn.gelu(o) + ft_ref[b]
        if split:
            out_ref[b] = o[:, :_N]
            outp_ref[b] = o[:, _N:]
        else:
            out_ref[b] = o
            if outp_ref is not None:
                outp_ref[b] = o.T


def _conv(ft, mx, w, b, g, be, want_pc):
    wint = w.reshape(_C, _C, 2)
    if want_pc:
        body = _conv_body
        out_shape = (jax.ShapeDtypeStruct((_B, _C, _P), jnp.float32),
                     jax.ShapeDtypeStruct((_B, _P, _C), jnp.float32))
    else:
        body = functools.partial(_conv_body, split=True)
        out_shape = (jax.ShapeDtypeStruct((_B, _C, _N), jnp.float32),
                     jax.ShapeDtypeStruct((_B, _C, _M), jnp.float32))
    return pl.pallas_call(
        body,
        out_shape=out_shape,
    )(ft, mx, wint[:, :, 0], wint[:, :, 1], b, g, be)


def kernel(x, y, W1, b1, g1, be1, W2, b2, g2, be2):
    ft0 = jnp.concatenate([x[..., 0], y[..., 0]], axis=2)
    ftp0 = jnp.swapaxes(ft0, 1, 2)
    dt = _dist(ft0)
    idxt = _sc_topk(dt)
    mx0 = _gather_max(ftp0, idxt)
    ft1, ftp1 = _conv(ft0, mx0, W1, b1, g1, be1, True)
    mx1 = _gather_max(ftp1, idxt)
    ox, oy = _conv(ft1, mx1, W2, b2, g2, be2, False)
    return ox[..., None], oy[..., None]


# speedup vs baseline: 1.0819x; 1.0113x over previous
import functools

import jax
import jax.numpy as jnp
from jax import lax
from jax.experimental import pallas as pl
from jax.experimental.pallas import tpu as pltpu
from jax.experimental.pallas import tpu_sc as plsc

_B, _C, _N, _M = 2, 256, 1024, 1024
_P = _N + _M
_K = 16
_EPS = 1e-5
_R = 256
_TPB = _P // 8 * (_P // 128)


def _dist_body(f_blk_ref, f_all_ref, d_ref):
    fb = f_blk_ref[0]
    fa = f_all_ref[0]
    g = lax.dot_general(fb, fa, (((0,), (0,)), ((), ())),
                        preferred_element_type=jnp.float32)
    c2 = jnp.sum(fa * fa, axis=0)
    d = c2[None, :] - 2.0 * g
    dt = d.reshape(_R // 8, 8, _P // 128, 128).transpose(0, 2, 1, 3)
    d_ref[0] = dt.reshape(_R // 8 * (_P // 128), 8, 128)


def _dist(ft):
    return pl.pallas_call(
        _dist_body,
        grid=(_B, _P // _R),
        in_specs=[
            pl.BlockSpec((1, _C, _R), lambda b, i: (b, 0, i)),
            pl.BlockSpec((1, _C, _P), lambda b, i: (b, 0, 0)),
        ],
        out_specs=pl.BlockSpec((1, _R // 8 * (_P // 128), 8, 128),
                               lambda b, i: (b, i, 0, 0)),
        out_shape=jax.ShapeDtypeStruct((_B, _TPB, 8, 128), jnp.float32),
    )(ft, ft)


def _sc_topk(dt):
    mesh = plsc.VectorSubcoreMesh(core_axis_name="c", subcore_axis_name="s")

    @functools.partial(
        pl.kernel,
        out_type=jax.ShapeDtypeStruct((_B, _P, 2 * _K), jnp.int32),
        mesh=mesh,
        scratch_types=[
            pltpu.VMEM((16, 8, 128), jnp.float32),
            pltpu.VMEM((16, 8, 128), jnp.float32),
            pltpu.VMEM((128, 2 * _K), jnp.int32),
            pltpu.SemaphoreType.DMA,
            pltpu.SemaphoreType.DMA,
        ],
        compiler_params=pltpu.CompilerParams(
            use_tc_tiling_on_sc=False, needs_layout_passes=False),
    )
    def tk(d_hbm, out_hbm, d_v0, d_v1, out_v, sem0, sem1):
        cid = lax.axis_index("c")
        sid = lax.axis_index("s")
        wid = sid * 2 + cid
        b = wid // 16
        rb0 = (wid % 16) * 16
        iota = lax.broadcasted_iota(jnp.int32, (16,), 0)

        def dcopy(blk, buf, sem):
            return pltpu.make_async_copy(
                d_hbm.at[b, pl.ds((rb0 + blk) * 16, 16)], buf, sem)

        def process(buf, blk):
            for grp in range(2):
                ks, vs = [], []
                for r in range(4 * grp, 4 * grp + 4):
                    for h in range(2):
                        k0 = buf[h * 8, r, pl.ds(0, 16)]
                        sk, sv = plsc.sort_key_val(k0, iota + h * 1024)
                        ks.append(sk)
                        vs.append(sv)

                def chunk(c, kv):
                    ks, vs = kv
                    nks, nvs = [], []
                    for hh in range(2):
                        c16 = hh * 64 + c
                        ci = c16 // 8
                        co = (c16 % 8) * 16
                        base = hh * 1024 + c * 16
                        for rr in range(4):
                            r = 4 * grp + rr
                            j = rr * 2 + hh
                            kc = buf[ci, r, pl.ds(co, 16)]
                            kcd, icd = plsc.sort_key_val(
                                kc, iota + base, descending=True)
                            sel = kcd < ks[j]
                            km = jnp.where(sel, kcd, ks[j])
                            vm = jnp.where(sel, icd, vs[j])
                            km, vm = plsc.sort_key_val(km, vm)
                            nks.append(km)
                            nvs.append(vm)
                    nks = [nks[hh * 4 + rr] for rr in range(4) for hh in range(2)]
                    nvs = [nvs[hh * 4 + rr] for rr in range(4) for hh in range(2)]
                    return tuple(nks), tuple(nvs)

                ks, vs = lax.fori_loop(1, 64, chunk, (tuple(ks), tuple(vs)))
                for rr in range(4):
                    r = 4 * grp + rr
                    for h in range(2):
                        out_v[blk * 8 + r, pl.ds(h * 16, 16)] = vs[rr * 2 + h]

        dcopy(0, d_v0, sem0).start()
        dcopy(1, d_v1, sem1).start()

        def pair(i, carry):
            blk0 = 2 * i
            dcopy(blk0, d_v0, sem0).wait()
            process(d_v0, blk0)

            @pl.when(i < 7)
            def _():
                dcopy(blk0 + 2, d_v0, sem0).start()

            dcopy(blk0 + 1, d_v1, sem1).wait()
            process(d_v1, blk0 + 1)

            @pl.when(i < 7)
            def _():
                dcopy(blk0 + 3, d_v1, sem1).start()

            return carry

        lax.fori_loop(0, 8, pair, 0)
        pltpu.sync_copy(out_v, out_hbm.at[b, pl.ds((wid % 16) * 128, 128)])

    return tk(dt)


_TILES_PER_B = 16
_CH = _C // _TILES_PER_B
_PH = _P // 2
_PC = 8


def _gather_max(ftp, idxt):
    mesh = plsc.VectorSubcoreMesh(core_axis_name="c", subcore_axis_name="s")

    @functools.partial(
        pl.kernel,
        out_type=jax.ShapeDtypeStruct((_B, _P // 8, _C // 128, 8, 128),
                                      jnp.float32),
        mesh=mesh,
        scratch_types=[
            pltpu.VMEM((_P, _CH), jnp.float32),
            pltpu.VMEM((_P, 2 * _K), jnp.int32),
            pltpu.VMEM((_PH // 8, 8, _CH), jnp.float32),
        ],
        compiler_params=pltpu.CompilerParams(
            use_tc_tiling_on_sc=False, needs_layout_passes=False),
    )
    def gm(ftp_hbm, idx_hbm, out_hbm, f_v, idx_v, out_v):
        cid = lax.axis_index("c")
        sid = lax.axis_index("s")
        wid = sid * 2 + cid
        b = wid // _TILES_PER_B
        c0 = (wid % _TILES_PER_B) * _CH
        pltpu.sync_copy(ftp_hbm.at[b, :, pl.ds(c0, _CH)], f_v)
        pltpu.sync_copy(idx_hbm.at[b], idx_v)
        for h in range(2):
            def chunk(i, carry):
                p0 = h * _PH + i * _PC
                ivs = [idx_v[p0 + k, pl.ds(j * 16, 16)]
                       for k in range(_PC) for j in range(2)]
                for k in range(_PC):
                    iv0 = ivs[2 * k]
                    iv1 = ivs[2 * k + 1]
                    m = f_v[iv0[0], :]
                    for l in range(1, 16):
                        m = jnp.maximum(m, f_v[iv0[l], :])
                    for l in range(16):
                        m = jnp.maximum(m, f_v[iv1[l], :])
                    q = i * _PC + k
                    out_v[q // 8, q % 8, :] = m
                return carry

            lax.fori_loop(0, _PH // _PC, chunk, 0)
            pltpu.sync_copy(
                out_v,
                out_hbm.at[b, pl.ds(h * (_PH // 8), _PH // 8), c0 // 128,
                           :, pl.ds(c0 % 128, _CH)])

    return gm(ftp, idxt)


def _conv_body(ft_ref, mx_ref, wa_ref, wb_ref, b_ref, g_ref, be_ref,
               out_ref, outp_ref, split=False):
    wa = wa_ref[...]
    wb = wb_ref[...]
    wd = wa - wb
    bias = b_ref[:]
    outs = []
    s1 = jnp.zeros((_C, 1), jnp.float32)
    for b in range(_B):
        mxp = mx_ref[b].transpose(0, 2, 1, 3).reshape(_P, _C)
        o = (jnp.dot(wd, ft_ref[b], preferred_element_type=jnp.float32)
             + lax.dot_general(wb, mxp, (((1,), (1,)), ((), ())),
                               preferred_element_type=jnp.float32))
        o = o + bias[:, None]
        outs.append(o)
        s1 = s1 + jnp.sum(o, axis=1, keepdims=True)
    mean = s1 * (1.0 / (_B * _P))
    s2 = jnp.zeros((_C, 1), jnp.float32)
    for b in range(_B):
        ctr = outs[b] - mean
        outs[b] = ctr
        s2 = s2 + jnp.sum(ctr * ctr, axis=1, keepdims=True)
    var = s2 * (1.0 / (_B * _P))
    scale = lax.rsqrt(var + _EPS) * g_ref[:][:, None]
    for b in range(_B):
        o = outs[b] * scale + be_ref[:][:, None]
        o = jax.nn.gelu(o) + ft_ref[b]
        if split:
            out_ref[b] = o[:, :_N]
            outp_ref[b] = o[:, _N:]
        else:
            out_ref[b] = o
            if outp_ref is not None:
                outp_ref[b] = o.T


def _conv(ft, mx, w, b, g, be, want_pc):
    wint = w.reshape(_C, _C, 2)
    if want_pc:
        body = _conv_body
        out_shape = (jax.ShapeDtypeStruct((_B, _C, _P), jnp.float32),
                     jax.ShapeDtypeStruct((_B, _P, _C), jnp.float32))
    else:
        body = functools.partial(_conv_body, split=True)
        out_shape = (jax.ShapeDtypeStruct((_B, _C, _N), jnp.float32),
                     jax.ShapeDtypeStruct((_B, _C, _M), jnp.float32))
    return pl.pallas_call(
        body,
        out_shape=out_shape,
    )(ft, mx, wint[:, :, 0], wint[:, :, 1], b, g, be)


def kernel(x, y, W1, b1, g1, be1, W2, b2, g2, be2):
    ft0 = jnp.concatenate([x[..., 0], y[..., 0]], axis=2)
    ftp0 = jnp.swapaxes(ft0, 1, 2)
    dt = _dist(ft0)
    idxt = _sc_topk(dt)
    mx0 = _gather_max(ftp0, idxt)
    ft1, ftp1 = _conv(ft0, mx0, W1, b1, g1, be1, True)
    mx1 = _gather_max(ftp1, idxt)
    ox, oy = _conv(ft1, mx1, W2, b2, g2, be2, False)
    return ox[..., None], oy[..., None]
